# baseline (speedup 1.0000x reference)
import functools

import jax
import jax.numpy as jnp
from jax import lax
from jax.experimental import pallas as pl
from jax.experimental.pallas import tpu as pltpu
from jax.experimental.pallas import tpu_sc as plsc

N_NODES = 100000
N_EDGES = 6400000
GAMMA = 0.1

_NC = 2
_NS = 16
NPAD = 100096
RPT = NPAD // _NS
ET = N_EDGES // (_NC * _NS)
CH = 4000
NIT = ET // CH
NV = CH // 16

_MESH = plsc.VectorSubcoreMesh(
    core_axis_name="c", subcore_axis_name="s", num_cores=_NC, num_subcores=_NS
)

_f32 = jnp.float32
_i32 = jnp.int32


def _lj_coef(drx, dry):
    r2 = drx * drx + dry * dry
    r2c = jnp.maximum(r2, _f32(1e-30))
    i = lax.bitcast_convert_type(r2c, _i32)
    i = _i32(0x5F3759DF) - (i >> 1)
    y = lax.bitcast_convert_type(i, _f32)
    for _ in range(3):
        y = y * (_f32(1.5) - _f32(0.5) * r2c * y * y)
    abs_dr = r2 * y
    inv_den = _f32(1.0) / jnp.maximum(abs_dr, _f32(1e-12))
    u = jnp.minimum(y, _f32(10.0))
    u2 = u * u
    u6 = u2 * u2 * u2
    lj = _f32(4.0) * u6 * (_f32(12.0) * u6 - _f32(6.0)) * u
    return lj * inv_den


@functools.partial(
    pl.kernel,
    out_type=[jax.ShapeDtypeStruct((2 * NPAD,), _f32) for _ in range(2)],
    mesh=_MESH,
    scratch_types=[
        [pltpu.VMEM((CH,), _i32) for _ in range(2)],
        [pltpu.VMEM((CH,), _i32) for _ in range(2)],
        [pltpu.VMEM((CH,), _f32) for _ in range(2)],
        [pltpu.VMEM((CH,), _f32) for _ in range(2)],
        [pltpu.VMEM((CH,), _f32) for _ in range(2)],
        [pltpu.VMEM((CH,), _f32) for _ in range(2)],
        pltpu.VMEM((RPT,), _f32),
        pltpu.VMEM_SHARED((NPAD,), _f32),
        pltpu.VMEM_SHARED((NPAD,), _f32),
        pltpu.VMEM_SHARED((NPAD,), _f32),
        pltpu.VMEM_SHARED((NPAD,), _f32),
        pltpu.SemaphoreType.DMA,
        pltpu.SemaphoreType.DMA,
    ],
)
def _sc_scatter(xx_hbm, xy_hbm, src_hbm, dst_hbm, outx_hbm, outy_hbm,
                sidx, didx, sxx, sxy, dxx, dxy, tbuf,
                xx_sh, xy_sh, accx_sh, accy_sh, semG, semI):
    c = lax.axis_index("c")
    s = lax.axis_index("s")
    row0 = pl.multiple_of(s * RPT, 8)

    pltpu.sync_copy(xx_hbm.at[pl.ds(row0, RPT)], tbuf)
    pltpu.sync_copy(tbuf, xx_sh.at[pl.ds(row0, RPT)])
    pltpu.sync_copy(xy_hbm.at[pl.ds(row0, RPT)], tbuf)
    pltpu.sync_copy(tbuf, xy_sh.at[pl.ds(row0, RPT)])

    def _zero(i, carry):
        tbuf[pl.ds(i * 16, 16)] = jnp.zeros((16,), _f32)
        return carry

    lax.fori_loop(0, RPT // 16, _zero, 0)
    pltpu.sync_copy(tbuf, accx_sh.at[pl.ds(row0, RPT)])
    pltpu.sync_copy(tbuf, accy_sh.at[pl.ds(row0, RPT)])
    plsc.subcore_barrier()

    ebase = (c * _NS + s) * ET

    def _fire_idx(it, b):
        base = pl.multiple_of(ebase + it * CH, 8)
        pltpu.async_copy(src_hbm.at[pl.ds(base, CH)], sidx[b], semI)
        pltpu.async_copy(dst_hbm.at[pl.ds(base, CH)], didx[b], semI)

    def _wait_idx(b):
        pltpu.make_async_copy(src_hbm.at[pl.ds(0, CH)], sidx[b], semI).wait()
        pltpu.make_async_copy(dst_hbm.at[pl.ds(0, CH)], didx[b], semI).wait()

    def _fire_gathers(b):
        pltpu.async_copy(xx_sh.at[sidx[b]], sxx[b], semG)
        pltpu.async_copy(xy_sh.at[sidx[b]], sxy[b], semG)
        pltpu.async_copy(xx_sh.at[didx[b]], dxx[b], semG)
        pltpu.async_copy(xy_sh.at[didx[b]], dxy[b], semG)

    def _wait_gathers(b):
        pltpu.make_async_copy(xx_sh.at[sidx[b]], sxx[b], semG).wait()
        pltpu.make_async_copy(xy_sh.at[sidx[b]], sxy[b], semG).wait()
        pltpu.make_async_copy(xx_sh.at[didx[b]], dxx[b], semG).wait()
        pltpu.make_async_copy(xy_sh.at[didx[b]], dxy[b], semG).wait()

    def _step(it, b, nb):
        @pl.when(it + 1 < NIT)
        def _():
            _wait_idx(nb)
            _fire_gathers(nb)

        _wait_gathers(b)

        def _vec(i, carry2):
            o = pl.ds(i * 16, 16)
            drx = dxx[b][o] - sxx[b][o]
            dry = dxy[b][o] - sxy[b][o]
            cc = _lj_coef(drx, dry)
            dxx[b][o] = cc * drx
            dxy[b][o] = cc * dry
            return carry2

        lax.fori_loop(0, NV, _vec, 0)
        pltpu.sync_copy(dxx[b], accx_sh.at[didx[b]], add=True)
        pltpu.sync_copy(dxy[b], accy_sh.at[didx[b]], add=True)

        @pl.when(it + 2 < NIT)
        def _():
            _fire_idx(it + 2, b)

    base0 = pl.multiple_of(ebase, 8)
    pltpu.sync_copy(src_hbm.at[pl.ds(base0, CH)], sidx[0])
    pltpu.sync_copy(dst_hbm.at[pl.ds(base0, CH)], didx[0])
    _fire_gathers(0)
    _fire_idx(1, 1)

    def _outer(g, carry):
        _step(g * 2, 0, 1)
        _step(g * 2 + 1, 1, 0)
        return carry

    lax.fori_loop(0, NIT // 2, _outer, 0)

    plsc.subcore_barrier()
    orow = pl.multiple_of(c * NPAD + row0, 8)
    pltpu.sync_copy(accx_sh.at[pl.ds(row0, RPT)], tbuf)
    pltpu.sync_copy(tbuf, outx_hbm.at[pl.ds(orow, RPT)])
    pltpu.sync_copy(accy_sh.at[pl.ds(row0, RPT)], tbuf)
    pltpu.sync_copy(tbuf, outy_hbm.at[pl.ds(orow, RPT)])


_ROWS = NPAD // 128


def _combine_body(px0, px1, py0, py1, vx, vy, ax, ay):
    ax[...] = px0[...] + px1[...] - _f32(GAMMA) * vx[...]
    ay[...] = py0[...] + py1[...] - _f32(GAMMA) * vy[...]


_combine = pl.pallas_call(
    _combine_body,
    out_shape=[jax.ShapeDtypeStruct((_ROWS, 128), _f32) for _ in range(2)],
)


def kernel(x, v, edge_index):
    pad = NPAD - N_NODES
    xxp = jnp.pad(x[:, 0], (0, pad))
    xyp = jnp.pad(x[:, 1], (0, pad))
    src = edge_index[0]
    dst = edge_index[1]
    px, py = _sc_scatter(xxp, xyp, src, dst)
    px0 = px[:NPAD].reshape(_ROWS, 128)
    px1 = px[NPAD:].reshape(_ROWS, 128)
    py0 = py[:NPAD].reshape(_ROWS, 128)
    py1 = py[NPAD:].reshape(_ROWS, 128)
    vx = jnp.pad(v[:, 0], (0, pad)).reshape(_ROWS, 128)
    vy = jnp.pad(v[:, 1], (0, pad)).reshape(_ROWS, 128)
    ax, ay = _combine(px0, px1, py0, py1, vx, vy)
    return jnp.stack([ax.reshape(-1)[:N_NODES], ay.reshape(-1)[:N_NODES]], axis=-1)


# speedup vs baseline: 1.4617x; 1.4617x over previous
import functools

import jax
import jax.numpy as jnp
from jax import lax
from jax.experimental import pallas as pl
from jax.experimental.pallas import tpu as pltpu
from jax.experimental.pallas import tpu_sc as plsc

N_NODES = 100000
N_EDGES = 6400000
GAMMA = 0.1

_NC = 2
_NS = 16
NPAD = 100096
RPT = NPAD // _NS
ET = N_EDGES // (_NC * _NS)
CH = 4000
NIT = ET // CH
NV = CH // 16

_MESH = plsc.VectorSubcoreMesh(
    core_axis_name="c", subcore_axis_name="s", num_cores=_NC, num_subcores=_NS
)

_f32 = jnp.float32
_i32 = jnp.int32


def _lj_coef(drx, dry):
    r2 = drx * drx + dry * dry
    r2c = jnp.maximum(r2, _f32(1e-30))
    i = lax.bitcast_convert_type(r2c, _i32)
    i = _i32(0x5F3759DF) - (i >> 1)
    y = lax.bitcast_convert_type(i, _f32)
    for _ in range(3):
        y = y * (_f32(1.5) - _f32(0.5) * r2c * y * y)
    abs_dr = r2 * y
    inv_den = _f32(1.0) / jnp.maximum(abs_dr, _f32(1e-12))
    u = jnp.minimum(y, _f32(10.0))
    u2 = u * u
    u6 = u2 * u2 * u2
    lj = _f32(4.0) * u6 * (_f32(12.0) * u6 - _f32(6.0)) * u
    return lj * inv_den


@functools.partial(
    pl.kernel,
    out_type=[jax.ShapeDtypeStruct((2 * NPAD,), _f32) for _ in range(2)],
    mesh=_MESH,
    scratch_types=[
        [pltpu.VMEM((CH,), _i32) for _ in range(2)],
        [pltpu.VMEM((CH,), _i32) for _ in range(2)],
        [pltpu.VMEM((CH,), _f32) for _ in range(2)],
        [pltpu.VMEM((CH,), _f32) for _ in range(2)],
        [pltpu.VMEM((CH,), _f32) for _ in range(2)],
        [pltpu.VMEM((CH,), _f32) for _ in range(2)],
        [pltpu.VMEM((CH,), _f32) for _ in range(2)],
        [pltpu.VMEM((CH,), _f32) for _ in range(2)],
        pltpu.VMEM((RPT,), _f32),
        pltpu.VMEM_SHARED((NPAD,), _f32),
        pltpu.VMEM_SHARED((NPAD,), _f32),
        pltpu.VMEM_SHARED((NPAD,), _f32),
        pltpu.VMEM_SHARED((NPAD,), _f32),
        pltpu.SemaphoreType.DMA,
        pltpu.SemaphoreType.DMA,
    ],
)
def _sc_scatter(xx_hbm, xy_hbm, src_hbm, dst_hbm, outx_hbm, outy_hbm,
                sidx, didx, sxx, sxy, dxx, dxy, mgx, mgy, tbuf,
                xx_sh, xy_sh, accx_sh, accy_sh, semG, semI):
    c = lax.axis_index("c")
    s = lax.axis_index("s")
    row0 = pl.multiple_of(s * RPT, 8)

    pltpu.sync_copy(xx_hbm.at[pl.ds(row0, RPT)], tbuf)
    pltpu.sync_copy(tbuf, xx_sh.at[pl.ds(row0, RPT)])
    pltpu.sync_copy(xy_hbm.at[pl.ds(row0, RPT)], tbuf)
    pltpu.sync_copy(tbuf, xy_sh.at[pl.ds(row0, RPT)])

    def _zero(i, carry):
        tbuf[pl.ds(i * 16, 16)] = jnp.zeros((16,), _f32)
        return carry

    lax.fori_loop(0, RPT // 16, _zero, 0)
    pltpu.sync_copy(tbuf, accx_sh.at[pl.ds(row0, RPT)])
    pltpu.sync_copy(tbuf, accy_sh.at[pl.ds(row0, RPT)])
    plsc.subcore_barrier()

    ebase = (c * _NS + s) * ET

    def _fire_idx(it, b):
        base = pl.multiple_of(ebase + it * CH, 8)
        pltpu.async_copy(src_hbm.at[pl.ds(base, CH)], sidx[b], semI)
        pltpu.async_copy(dst_hbm.at[pl.ds(base, CH)], didx[b], semI)

    def _wait_idx(b):
        pltpu.make_async_copy(src_hbm.at[pl.ds(0, CH)], sidx[b], semI).wait()
        pltpu.make_async_copy(dst_hbm.at[pl.ds(0, CH)], didx[b], semI).wait()

    def _fire_gathers(b):
        pltpu.async_copy(xx_sh.at[sidx[b]], sxx[b], semG)
        pltpu.async_copy(xy_sh.at[sidx[b]], sxy[b], semG)
        pltpu.async_copy(xx_sh.at[didx[b]], dxx[b], semG)
        pltpu.async_copy(xy_sh.at[didx[b]], dxy[b], semG)

    def _wait_gathers(b):
        pltpu.make_async_copy(xx_sh.at[sidx[b]], sxx[b], semG).wait()
        pltpu.make_async_copy(xy_sh.at[sidx[b]], sxy[b], semG).wait()
        pltpu.make_async_copy(xx_sh.at[didx[b]], dxx[b], semG).wait()
        pltpu.make_async_copy(xy_sh.at[didx[b]], dxy[b], semG).wait()

    def _step(it, b, nb):
        @pl.when(it + 1 < NIT)
        def _():
            _wait_idx(nb)
            _fire_gathers(nb)

        _wait_gathers(b)

        def _vec(i, carry2):
            o = pl.ds(i * 16, 16)
            drx = dxx[b][o] - sxx[b][o]
            dry = dxy[b][o] - sxy[b][o]
            cc = _lj_coef(drx, dry)
            mgx[b][o] = cc * drx
            mgy[b][o] = cc * dry
            return carry2

        lax.fori_loop(0, NV, _vec, 0)
        pltpu.sync_copy(mgx[b], accx_sh.at[didx[b]], add=True)
        pltpu.sync_copy(mgy[b], accy_sh.at[didx[b]], add=True)

        @pl.when(it + 2 < NIT)
        def _():
            _fire_idx(it + 2, b)

    base0 = pl.multiple_of(ebase, 8)
    pltpu.sync_copy(src_hbm.at[pl.ds(base0, CH)], sidx[0])
    pltpu.sync_copy(dst_hbm.at[pl.ds(base0, CH)], didx[0])
    _fire_gathers(0)
    _fire_idx(1, 1)

    def _outer(g, carry):
        _step(g * 2, 0, 1)
        _step(g * 2 + 1, 1, 0)
        return carry

    lax.fori_loop(0, NIT // 2, _outer, 0)

    plsc.subcore_barrier()
    orow = pl.multiple_of(c * NPAD + row0, 8)
    pltpu.sync_copy(accx_sh.at[pl.ds(row0, RPT)], tbuf)
    pltpu.sync_copy(tbuf, outx_hbm.at[pl.ds(orow, RPT)])
    pltpu.sync_copy(accy_sh.at[pl.ds(row0, RPT)], tbuf)
    pltpu.sync_copy(tbuf, outy_hbm.at[pl.ds(orow, RPT)])


_ROWS = NPAD // 128


def _combine_body(px0, px1, py0, py1, vx, vy, ax, ay):
    ax[...] = px0[...] + px1[...] - _f32(GAMMA) * vx[...]
    ay[...] = py0[...] + py1[...] - _f32(GAMMA) * vy[...]


_combine = pl.pallas_call(
    _combine_body,
    out_shape=[jax.ShapeDtypeStruct((_ROWS, 128), _f32) for _ in range(2)],
)


def kernel(x, v, edge_index):
    pad = NPAD - N_NODES
    xxp = jnp.pad(x[:, 0], (0, pad))
    xyp = jnp.pad(x[:, 1], (0, pad))
    src = edge_index[0]
    dst = edge_index[1]
    px, py = _sc_scatter(xxp, xyp, src, dst)
    px0 = px[:NPAD].reshape(_ROWS, 128)
    px1 = px[NPAD:].reshape(_ROWS, 128)
    py0 = py[:NPAD].reshape(_ROWS, 128)
    py1 = py[NPAD:].reshape(_ROWS, 128)
    vx = jnp.pad(v[:, 0], (0, pad)).reshape(_ROWS, 128)
    vy = jnp.pad(v[:, 1], (0, pad)).reshape(_ROWS, 128)
    ax, ay = _combine(px0, px1, py0, py1, vx, vy)
    return jnp.stack([ax.reshape(-1)[:N_NODES], ay.reshape(-1)[:N_NODES]], axis=-1)


# speedup vs baseline: 1.4851x; 1.0160x over previous
import functools

import jax
import jax.numpy as jnp
from jax import lax
from jax.experimental import pallas as pl
from jax.experimental.pallas import tpu as pltpu
from jax.experimental.pallas import tpu_sc as plsc

N_NODES = 100000
N_EDGES = 6400000
GAMMA = 0.1

_NC = 2
_NS = 16
NPAD = 100096
RPT = NPAD // _NS
ET = N_EDGES // (_NC * _NS)
CH = 4000
NIT = ET // CH
NV = CH // 16

_MESH = plsc.VectorSubcoreMesh(
    core_axis_name="c", subcore_axis_name="s", num_cores=_NC, num_subcores=_NS
)

_f32 = jnp.float32
_i32 = jnp.int32


def _lj_coef(drx, dry):
    r2 = drx * drx + dry * dry
    r2c = jnp.maximum(r2, _f32(1e-30))
    i = lax.bitcast_convert_type(r2c, _i32)
    i = _i32(0x5F3759DF) - (i >> 1)
    y = lax.bitcast_convert_type(i, _f32)
    for _ in range(3):
        y = y * (_f32(1.5) - _f32(0.5) * r2c * y * y)
    abs_dr = r2 * y
    inv_den = _f32(1.0) / jnp.maximum(abs_dr, _f32(1e-12))
    u = jnp.minimum(y, _f32(10.0))
    u2 = u * u
    u6 = u2 * u2 * u2
    lj = _f32(4.0) * u6 * (_f32(12.0) * u6 - _f32(6.0)) * u
    return lj * inv_den


@functools.partial(
    pl.kernel,
    out_type=[jax.ShapeDtypeStruct((2 * NPAD,), _f32) for _ in range(2)],
    mesh=_MESH,
    scratch_types=[
        [pltpu.VMEM((CH,), _i32) for _ in range(2)],
        [pltpu.VMEM((CH,), _i32) for _ in range(2)],
        [pltpu.VMEM((CH,), _f32) for _ in range(2)],
        [pltpu.VMEM((CH,), _f32) for _ in range(2)],
        [pltpu.VMEM((CH,), _f32) for _ in range(2)],
        [pltpu.VMEM((CH,), _f32) for _ in range(2)],
        [pltpu.VMEM((CH,), _f32) for _ in range(2)],
        [pltpu.VMEM((CH,), _f32) for _ in range(2)],
        [pltpu.VMEM((CH,), _i32) for _ in range(2)],
        pltpu.VMEM((RPT,), _f32),
        pltpu.VMEM_SHARED((NPAD,), _f32),
        pltpu.VMEM_SHARED((NPAD,), _f32),
        pltpu.VMEM_SHARED((NPAD,), _f32),
        pltpu.VMEM_SHARED((NPAD,), _f32),
        pltpu.SemaphoreType.DMA,
        pltpu.SemaphoreType.DMA,
        pltpu.SemaphoreType.DMA,
    ],
)
def _sc_scatter(xx_hbm, xy_hbm, src_hbm, dst_hbm, outx_hbm, outy_hbm,
                sidx, didx, sxx, sxy, dxx, dxy, mgx, mgy, dscat, tbuf,
                xx_sh, xy_sh, accx_sh, accy_sh, semG, semI, semS):
    c = lax.axis_index("c")
    s = lax.axis_index("s")
    row0 = pl.multiple_of(s * RPT, 8)

    pltpu.sync_copy(xx_hbm.at[pl.ds(row0, RPT)], tbuf)
    pltpu.sync_copy(tbuf, xx_sh.at[pl.ds(row0, RPT)])
    pltpu.sync_copy(xy_hbm.at[pl.ds(row0, RPT)], tbuf)
    pltpu.sync_copy(tbuf, xy_sh.at[pl.ds(row0, RPT)])

    def _zero(i, carry):
        tbuf[pl.ds(i * 16, 16)] = jnp.zeros((16,), _f32)
        return carry

    lax.fori_loop(0, RPT // 16, _zero, 0)
    pltpu.sync_copy(tbuf, accx_sh.at[pl.ds(row0, RPT)])
    pltpu.sync_copy(tbuf, accy_sh.at[pl.ds(row0, RPT)])
    plsc.subcore_barrier()

    ebase = (c * _NS + s) * ET

    def _fire_idx(it, b):
        base = pl.multiple_of(ebase + it * CH, 8)
        pltpu.async_copy(src_hbm.at[pl.ds(base, CH)], sidx[b], semI)
        pltpu.async_copy(dst_hbm.at[pl.ds(base, CH)], didx[b], semI)

    def _wait_idx(b):
        pltpu.make_async_copy(src_hbm.at[pl.ds(0, CH)], sidx[b], semI).wait()
        pltpu.make_async_copy(dst_hbm.at[pl.ds(0, CH)], didx[b], semI).wait()

    def _fire_gathers(b):
        pltpu.async_copy(xx_sh.at[sidx[b]], sxx[b], semG)
        pltpu.async_copy(xy_sh.at[sidx[b]], sxy[b], semG)
        pltpu.async_copy(xx_sh.at[didx[b]], dxx[b], semG)
        pltpu.async_copy(xy_sh.at[didx[b]], dxy[b], semG)

    def _wait_gathers(b):
        pltpu.make_async_copy(xx_sh.at[sidx[b]], sxx[b], semG).wait()
        pltpu.make_async_copy(xy_sh.at[sidx[b]], sxy[b], semG).wait()
        pltpu.make_async_copy(xx_sh.at[didx[b]], dxx[b], semG).wait()
        pltpu.make_async_copy(xy_sh.at[didx[b]], dxy[b], semG).wait()

    def _step(it, b, nb):
        @pl.when(it + 1 < NIT)
        def _():
            _wait_idx(nb)
            _fire_gathers(nb)

        _wait_gathers(b)

        def _vec(i, carry2):
            o = pl.ds(i * 16, 16)
            drx = dxx[b][o] - sxx[b][o]
            dry = dxy[b][o] - sxy[b][o]
            cc = _lj_coef(drx, dry)
            mgx[b][o] = cc * drx
            mgy[b][o] = cc * dry
            dscat[b][o] = didx[b][o]
            return carry2

        lax.fori_loop(0, NV, _vec, 0)

        @pl.when(it > 0)
        def _():
            pltpu.make_async_copy(mgx[nb], accx_sh.at[dscat[nb]], semS).wait()
            pltpu.make_async_copy(mgy[nb], accy_sh.at[dscat[nb]], semS).wait()

        pltpu.async_copy(mgx[b], accx_sh.at[dscat[b]], semS, add=True)
        pltpu.async_copy(mgy[b], accy_sh.at[dscat[b]], semS, add=True)

        @pl.when(it + 2 < NIT)
        def _():
            _fire_idx(it + 2, b)

    base0 = pl.multiple_of(ebase, 8)
    pltpu.sync_copy(src_hbm.at[pl.ds(base0, CH)], sidx[0])
    pltpu.sync_copy(dst_hbm.at[pl.ds(base0, CH)], didx[0])
    _fire_gathers(0)
    _fire_idx(1, 1)

    def _outer(g, carry):
        _step(g * 2, 0, 1)
        _step(g * 2 + 1, 1, 0)
        return carry

    lax.fori_loop(0, NIT // 2, _outer, 0)
    _fb = (NIT - 1) % 2
    pltpu.make_async_copy(mgx[_fb], accx_sh.at[dscat[_fb]], semS).wait()
    pltpu.make_async_copy(mgy[_fb], accy_sh.at[dscat[_fb]], semS).wait()

    plsc.subcore_barrier()
    orow = pl.multiple_of(c * NPAD + row0, 8)
    pltpu.sync_copy(accx_sh.at[pl.ds(row0, RPT)], tbuf)
    pltpu.sync_copy(tbuf, outx_hbm.at[pl.ds(orow, RPT)])
    pltpu.sync_copy(accy_sh.at[pl.ds(row0, RPT)], tbuf)
    pltpu.sync_copy(tbuf, outy_hbm.at[pl.ds(orow, RPT)])


_ROWS = NPAD // 128


def _combine_body(px0, px1, py0, py1, vx, vy, ax, ay):
    ax[...] = px0[...] + px1[...] - _f32(GAMMA) * vx[...]
    ay[...] = py0[...] + py1[...] - _f32(GAMMA) * vy[...]


_combine = pl.pallas_call(
    _combine_body,
    out_shape=[jax.ShapeDtypeStruct((_ROWS, 128), _f32) for _ in range(2)],
)


def kernel(x, v, edge_index):
    pad = NPAD - N_NODES
    xxp = jnp.pad(x[:, 0], (0, pad))
    xyp = jnp.pad(x[:, 1], (0, pad))
    src = edge_index[0]
    dst = edge_index[1]
    px, py = _sc_scatter(xxp, xyp, src, dst)
    px0 = px[:NPAD].reshape(_ROWS, 128)
    px1 = px[NPAD:].reshape(_ROWS, 128)
    py0 = py[:NPAD].reshape(_ROWS, 128)
    py1 = py[NPAD:].reshape(_ROWS, 128)
    vx = jnp.pad(v[:, 0], (0, pad)).reshape(_ROWS, 128)
    vy = jnp.pad(v[:, 1], (0, pad)).reshape(_ROWS, 128)
    ax, ay = _combine(px0, px1, py0, py1, vx, vy)
    return jnp.stack([ax.reshape(-1)[:N_NODES], ay.reshape(-1)[:N_NODES]], axis=-1)
